# baseline (speedup 1.0000x reference)
import dataclasses
import functools

import jax
import jax.numpy as jnp
from jax import lax
from jax.experimental import pallas as pl
from jax.experimental.pallas import tpu as pltpu
from jax.experimental.pallas import tpu_sc as plsc

N_USERS = 25000
N_ITEMS = 25000
N = N_USERS + N_ITEMS
D = 32
N_LAYERS = 3
NNZ = 1600000

NC = 2
NS = 16
L = 16
NW = NC * NS

CHUNK = 80
CROWS = NNZ // CHUNK
CROWS_PER_W = CROWS // NW
K = 25
NUM_BLOCKS = CROWS_PER_W // K
NBUF = 5
N_PAD = 50176
ROWS_PER_SUB = N_PAD // NS
CSLICE = N_PAD // NW
CC = 56
NCC = CSLICE // CC

_MESH = plsc.VectorSubcoreMesh(core_axis_name="c", subcore_axis_name="s")

_CP = pltpu.CompilerParams(use_tc_tiling_on_sc=False)
if "needs_layout_passes" in pltpu.CompilerParams.__dataclass_fields__:
    _CP = dataclasses.replace(_CP, needs_layout_passes=False)


def _fused_body(row_hbm, col_hbm, val_hbm, emb_hbm,
                tot_hbm, src_hbm, par_hbm,
                cb0, cb1, rb0, rb1, vb0, vb1,
                g0, g1, g2, g3, g4, g5, acc,
                gs0, gs1, gs2, gs3, gs4, ss0, ss1, ss2, ss3, ss4,
                is0, is1, zsem, bsem):
    cid = lax.axis_index("c")
    sid = lax.axis_index("s")
    wid = cid * NS + sid

    colb = (cb0, cb1)
    rowb = (rb0, rb1)
    valb = (vb0, vb1)
    gbuf = (g0, g1, g2, g3, g4, g5)
    gsem = (gs0, gs1, gs2, gs3, gs4)
    ssem = (ss0, ss1, ss2, ss3, ss4)
    isem = (is0, is1)

    zero = jnp.zeros((L,), jnp.float32)
    abase = sid * ROWS_PER_SUB
    nzf = ROWS_PER_SUB // CHUNK
    zrem = ROWS_PER_SUB - nzf * CHUNK
    cbase = wid * CSLICE

    def idx_issue0():
        cb = wid * CROWS_PER_W
        pltpu.async_copy(row_hbm.at[pl.ds(cb, K)], rowb[0], zsem)
        pltpu.async_copy(col_hbm.at[pl.ds(cb, K)], colb[0], zsem)
        pltpu.async_copy(val_hbm.at[pl.ds(cb, K)], valb[0], zsem)

    def zero_issue():
        g = gbuf[0]

        @pl.loop(0, CHUNK)
        def _(i):
            g[i, pl.ds(0, L)] = zero
            g[i, pl.ds(L, L)] = zero

        for k in range(nzf):
            pltpu.async_copy(g, acc.at[pl.ds(abase + k * CHUNK, CHUNK)], zsem)
        pltpu.async_copy(g.at[pl.ds(0, zrem)],
                         acc.at[pl.ds(abase + nzf * CHUNK, zrem)], zsem)

    def zero_idx_drain():
        pltpu.make_async_copy(row_hbm.at[pl.ds(0, K)], rowb[0], zsem).wait()
        pltpu.make_async_copy(col_hbm.at[pl.ds(0, K)], colb[0], zsem).wait()
        pltpu.make_async_copy(val_hbm.at[pl.ds(0, K)], valb[0], zsem).wait()
        for k in range(nzf):
            pltpu.make_async_copy(gbuf[0], acc.at[pl.ds(abase, CHUNK)],
                                  zsem).wait()
        pltpu.make_async_copy(gbuf[0].at[pl.ds(0, zrem)],
                              acc.at[pl.ds(abase, zrem)], zsem).wait()

    def edge_phase():
        def gissue(p, c, b):
            pltpu.async_copy(src_hbm.at[colb[p].at[c]], gbuf[b], gsem[b])

        def gwait(b):
            pltpu.make_async_copy(src_hbm.at[colb[0].at[0]], gbuf[b],
                                  gsem[b]).wait()

        def sissue(p, c, b):
            pltpu.async_copy(gbuf[b], acc.at[rowb[p].at[c]], ssem[b], add=True)

        def swait(b):
            pltpu.make_async_copy(gbuf[b], acc.at[rowb[0].at[0]],
                                  ssem[b]).wait()

        def iissue(p, blk):
            cb = wid * CROWS_PER_W + blk * K
            pltpu.async_copy(row_hbm.at[pl.ds(cb, K)], rowb[p], isem[p])
            pltpu.async_copy(col_hbm.at[pl.ds(cb, K)], colb[p], isem[p])
            pltpu.async_copy(val_hbm.at[pl.ds(cb, K)], valb[p], isem[p])

        def iwait(p):
            pltpu.make_async_copy(row_hbm.at[pl.ds(0, K)], rowb[p],
                                  isem[p]).wait()
            pltpu.make_async_copy(col_hbm.at[pl.ds(0, K)], colb[p],
                                  isem[p]).wait()
            pltpu.make_async_copy(val_hbm.at[pl.ds(0, K)], valb[p],
                                  isem[p]).wait()

        def mul(p, c, b):
            g = gbuf[b]
            vb = valb[p]

            @plsc.parallel_loop(0, CHUNK, step=L, unroll=2)
            def _(e0):
                vv = vb[c, pl.ds(e0, L)]
                for i in range(L):
                    v = vv.at[jnp.full((L,), i, jnp.int32)].get(
                        mode="promise_in_bounds")
                    e = e0 + i
                    g[e, pl.ds(0, L)] = g[e, pl.ds(0, L)] * v
                    g[e, pl.ds(L, L)] = g[e, pl.ds(L, L)] * v

        def body(blk, p, last):
            if not last:
                iissue(1 - p, blk + 1)

            @pl.loop(0, K - NBUF, step=NBUF)
            def _(c0):
                for b in range(NBUF):
                    gwait(b)
                    mul(p, c0 + b, b)
                    sissue(p, c0 + b, b)
                for b in range(NBUF):
                    swait(b)
                    gissue(p, c0 + NBUF + b, b)

            for b in range(NBUF):
                gwait(b)
                mul(p, K - NBUF + b, b)
                sissue(p, K - NBUF + b, b)
            if not last:
                iwait(1 - p)
                for b in range(NBUF):
                    swait(b)
                    gissue(1 - p, b, b)
            else:
                for b in range(NBUF):
                    swait(b)

        for b in range(NBUF):
            gissue(0, b, b)

        @pl.loop(0, NUM_BLOCKS - 1, step=2)
        def _(blk):
            body(blk, 0, False)
            body(blk + 1, 1, False)

        body(NUM_BLOCKS - 1, 0, True)

    own = (gbuf[0], gbuf[1])
    oth = (gbuf[2], gbuf[3])
    tob = (gbuf[4], gbuf[5])

    def combine():
        po = par_hbm.at[1 - cid]

        def cread(i, q):
            off = cbase + i * CC
            pltpu.async_copy(acc.at[pl.ds(off, CC)],
                             own[q].at[pl.ds(0, CC)], gsem[q])
            pltpu.async_copy(po.at[pl.ds(off, CC)],
                             oth[q].at[pl.ds(0, CC)], gsem[2 + q])
            pltpu.async_copy(tot_hbm.at[pl.ds(off, CC)],
                             tob[q].at[pl.ds(0, CC)], ssem[q])

        def cwait_read(q):
            pltpu.make_async_copy(acc.at[pl.ds(0, CC)],
                                  own[q].at[pl.ds(0, CC)], gsem[q]).wait()
            pltpu.make_async_copy(po.at[pl.ds(0, CC)],
                                  oth[q].at[pl.ds(0, CC)], gsem[2 + q]).wait()
            pltpu.make_async_copy(tot_hbm.at[pl.ds(0, CC)],
                                  tob[q].at[pl.ds(0, CC)], ssem[q]).wait()

        def ccompute(q):
            go, gt, gb = own[q], oth[q], tob[q]

            @plsc.parallel_loop(0, CC, step=1, unroll=2)
            def _(r):
                for h in (0, L):
                    e = go[r, pl.ds(h, L)] + gt[r, pl.ds(h, L)]
                    t = gb[r, pl.ds(h, L)] + e
                    go[r, pl.ds(h, L)] = e
                    gb[r, pl.ds(h, L)] = t

        def cwrite(i, q):
            off = cbase + i * CC
            pltpu.async_copy(tob[q].at[pl.ds(0, CC)],
                             tot_hbm.at[pl.ds(off, CC)], ssem[2 + q])
            pltpu.async_copy(own[q].at[pl.ds(0, CC)],
                             src_hbm.at[pl.ds(off, CC)], isem[q])

        def cwait_write(q):
            pltpu.make_async_copy(tob[q].at[pl.ds(0, CC)],
                                  tot_hbm.at[pl.ds(0, CC)], ssem[2 + q]).wait()
            pltpu.make_async_copy(own[q].at[pl.ds(0, CC)],
                                  src_hbm.at[pl.ds(0, CC)], isem[q]).wait()

        def cbody(i, q):
            cwait_read(q)
            ccompute(q)
            cwrite(i, q)
            cwait_write(q)
            cread(i + 2, q)

        cread(0, 0)
        cread(1, 1)

        @pl.loop(0, NCC - 2, step=2)
        def _(i):
            cbody(i, 0)
            cbody(i + 1, 1)

        for i, q in ((NCC - 2, 0), (NCC - 1, 1)):
            cwait_read(q)
            ccompute(q)
            cwrite(i, q)
            cwait_write(q)

    idx_issue0()
    zero_issue()
    pltpu.async_copy(emb_hbm.at[pl.ds(cbase, CSLICE)],
                     src_hbm.at[pl.ds(cbase, CSLICE)], isem[0])
    pltpu.async_copy(emb_hbm.at[pl.ds(cbase, CSLICE)],
                     tot_hbm.at[pl.ds(cbase, CSLICE)], isem[1])
    zero_idx_drain()
    pltpu.make_async_copy(emb_hbm.at[pl.ds(0, CSLICE)],
                          src_hbm.at[pl.ds(0, CSLICE)], isem[0]).wait()
    pltpu.make_async_copy(emb_hbm.at[pl.ds(0, CSLICE)],
                          tot_hbm.at[pl.ds(0, CSLICE)], isem[1]).wait()
    plsc.subcore_barrier()
    pltpu.core_barrier(bsem, core_axis_name="c")

    @pl.loop(0, N_LAYERS)
    def _(l):
        edge_phase()
        plsc.subcore_barrier()
        pltpu.sync_copy(acc.at[pl.ds(abase, ROWS_PER_SUB)],
                        par_hbm.at[cid].at[pl.ds(abase, ROWS_PER_SUB)])
        plsc.subcore_barrier()
        pltpu.core_barrier(bsem, core_axis_name="c")
        idx_issue0()
        combine()
        plsc.subcore_barrier()
        pltpu.core_barrier(bsem, core_axis_name="c")
        zero_issue()
        zero_idx_drain()
        plsc.subcore_barrier()


@functools.partial(
    pl.kernel,
    out_type=[
        jax.ShapeDtypeStruct((N_PAD, D), jnp.float32),
        jax.ShapeDtypeStruct((N_PAD, D), jnp.float32),
        jax.ShapeDtypeStruct((NC, N_PAD, D), jnp.float32),
    ],
    mesh=_MESH,
    scratch_types=(
        [pltpu.VMEM((K, CHUNK), jnp.int32)] * 2
        + [pltpu.VMEM((K, CHUNK), jnp.int32)] * 2
        + [pltpu.VMEM((K, CHUNK), jnp.float32)] * 2
        + [pltpu.VMEM((CHUNK, D), jnp.float32)] * 6
        + [pltpu.VMEM_SHARED((N_PAD, D), jnp.float32)]
        + [pltpu.SemaphoreType.DMA] * (2 * NBUF + 3)
        + [pltpu.SemaphoreType.REGULAR]
    ),
    compiler_params=_CP,
)
def _lightgcn(row_hbm, col_hbm, val_hbm, emb_hbm, *rest):
    _fused_body(row_hbm, col_hbm, val_hbm, emb_hbm, *rest)


def kernel(adj_indices, adj_values, user_emb, item_emb):
    row = adj_indices[0].reshape(CROWS, CHUNK)
    col = adj_indices[1].reshape(CROWS, CHUNK)
    val = adj_values.reshape(CROWS, CHUNK)
    emb = jnp.concatenate(
        [user_emb, item_emb, jnp.zeros((N_PAD - N, D), jnp.float32)], axis=0)

    total, _, _ = _lightgcn(row, col, val, emb)
    final = total * (1.0 / (N_LAYERS + 1))
    return final[:N_USERS], final[N_USERS:N]
